# baseline (speedup 1.0000x reference)
import functools

import jax
import jax.numpy as jnp
from jax import lax
from jax.experimental import pallas as pl
from jax.experimental.pallas import tpu as pltpu
from jax.experimental.pallas import tpu_sc as plsc

EMBED_DIM = 128
BATCH = 16384

NC = 2
NS = 16
L = 16
NW = NC * NS
B_PER_W = BATCH // NW
CHUNK = 64
N_CHUNKS = B_PER_W // CHUNK
NVEC = EMBED_DIM // L
GROUPS = CHUNK // L
NBUF = 6
PITCH = L + 1


@functools.partial(
    pl.kernel,
    out_type=jax.ShapeDtypeStruct((BATCH,), jnp.float32),
    mesh=plsc.VectorSubcoreMesh(core_axis_name="c", subcore_axis_name="s"),
    compiler_params=pltpu.CompilerParams(needs_layout_passes=False),
    scratch_types=[
        pltpu.VMEM((B_PER_W,), jnp.int32),
        pltpu.VMEM((B_PER_W,), jnp.int32),
        pltpu.VMEM((NBUF, CHUNK, EMBED_DIM), jnp.float32),
        pltpu.VMEM((NBUF, CHUNK, EMBED_DIM), jnp.float32),
        pltpu.VMEM((L, PITCH), jnp.float32),
        pltpu.VMEM((L, PITCH), jnp.float32),
        pltpu.VMEM((B_PER_W,), jnp.float32),
        pltpu.SemaphoreType.DMA((NBUF,)),
        pltpu.SemaphoreType.DMA((NBUF,)),
        pltpu.SemaphoreType.DMA,
    ],
)
def _skipgram_sc(tgt_hbm, ctx_hbm, emb_hbm, ow_hbm, out_hbm,
                 tgt_v, ctx_v, e_buf, w_buf, stg_a, stg_b, out_v,
                 sem_e, sem_w, sem_o):
    wid = lax.axis_index("s") * NC + lax.axis_index("c")
    base = wid * B_PER_W

    lane = lax.iota(jnp.int32, L)

    def copy_pair(k, p):
        return (
            pltpu.make_async_copy(
                emb_hbm.at[tgt_v.at[pl.ds(k * CHUNK, CHUNK)]],
                e_buf.at[p], sem_e.at[p]),
            pltpu.make_async_copy(
                ow_hbm.at[ctx_v.at[pl.ds(k * CHUNK, CHUNK)]],
                w_buf.at[p], sem_w.at[p]),
        )

    rest = B_PER_W - CHUNK
    it0 = pltpu.make_async_copy(
        tgt_hbm.at[pl.ds(base, CHUNK)], tgt_v.at[pl.ds(0, CHUNK)],
        sem_e.at[NBUF - 1])
    ic0 = pltpu.make_async_copy(
        ctx_hbm.at[pl.ds(base, CHUNK)], ctx_v.at[pl.ds(0, CHUNK)],
        sem_w.at[NBUF - 1])
    it0.start()
    ic0.start()
    itr = pltpu.make_async_copy(
        tgt_hbm.at[pl.ds(base + CHUNK, rest)], tgt_v.at[pl.ds(CHUNK, rest)],
        sem_e.at[NBUF - 2])
    icr = pltpu.make_async_copy(
        ctx_hbm.at[pl.ds(base + CHUNK, rest)], ctx_v.at[pl.ds(CHUNK, rest)],
        sem_w.at[NBUF - 2])
    itr.start()
    icr.start()
    it0.wait()
    ic0.wait()
    ce0, cw0 = copy_pair(0, 0)
    ce0.start()
    cw0.start()
    itr.wait()
    icr.wait()
    for k in range(1, min(NBUF, N_CHUNKS)):
        ce, cw = copy_pair(k, k)
        ce.start()
        cw.start()

    def chunk_step(k, carry):
        p = lax.rem(k, NBUF)
        ce, cw = copy_pair(k, p)
        ce.wait()
        cw.wait()

        e_b = e_buf.at[p]
        w_b = w_buf.at[p]

        def do_group(g, stg):
            rbase = g * L
            for sub in range(L // 4):
                paccs = []
                for rr in range(4):
                    row = rbase + sub * 4 + rr
                    prods = [e_b[row, pl.ds(c * L, L)] * w_b[row, pl.ds(c * L, L)]
                             for c in range(NVEC)]
                    while len(prods) > 1:
                        prods = [a + b for a, b in zip(prods[::2], prods[1::2])]
                    paccs.append(prods[0])
                for rr in range(4):
                    plsc.store_scatter(
                        stg,
                        [lane, jnp.full((L,), sub * 4 + rr, jnp.int32)],
                        paccs[rr])
            sums = [stg[j, pl.ds(0, L)] for j in range(L)]
            while len(sums) > 1:
                sums = [a + b for a, b in zip(sums[::2], sums[1::2])]
            out_v[pl.ds(k * CHUNK + rbase, L)] = sums[0]

        def group_pair(gp, carry2):
            do_group(gp * 2, stg_a)
            do_group(gp * 2 + 1, stg_b)
            return carry2

        lax.fori_loop(0, GROUPS // 2, group_pair, 0)

        @pl.when(k + NBUF < N_CHUNKS)
        def _():
            cne, cnw = copy_pair(k + NBUF, p)
            cne.start()
            cnw.start()

        pltpu.make_async_copy(
            out_v.at[pl.ds(k * CHUNK, CHUNK)],
            out_hbm.at[pl.ds(base + k * CHUNK, CHUNK)], sem_o).start()

        return carry

    lax.fori_loop(0, N_CHUNKS, chunk_step, 0)

    for k in range(N_CHUNKS):
        pltpu.make_async_copy(
            out_v.at[pl.ds(k * CHUNK, CHUNK)],
            out_hbm.at[pl.ds(base + k * CHUNK, CHUNK)], sem_o).wait()


def kernel(target, context, embeddings, output_weights):
    return _skipgram_sc(target, context, embeddings, output_weights)


# speedup vs baseline: 1.0047x; 1.0047x over previous
import functools

import jax
import jax.numpy as jnp
from jax import lax
from jax.experimental import pallas as pl
from jax.experimental.pallas import tpu as pltpu
from jax.experimental.pallas import tpu_sc as plsc

EMBED_DIM = 128
BATCH = 16384

NC = 2
NS = 16
L = 16
NW = NC * NS
B_PER_W = BATCH // NW
CHUNK = 64
N_CHUNKS = B_PER_W // CHUNK
NVEC = EMBED_DIM // L
GROUPS = CHUNK // L
NBUF = 6
PITCH = L + 1


@functools.partial(
    pl.kernel,
    out_type=jax.ShapeDtypeStruct((BATCH,), jnp.float32),
    mesh=plsc.VectorSubcoreMesh(core_axis_name="c", subcore_axis_name="s"),
    compiler_params=pltpu.CompilerParams(needs_layout_passes=False),
    scratch_types=[
        pltpu.VMEM((B_PER_W,), jnp.int32),
        pltpu.VMEM((B_PER_W,), jnp.int32),
        pltpu.VMEM((NBUF, CHUNK, EMBED_DIM), jnp.float32),
        pltpu.VMEM((NBUF, CHUNK, EMBED_DIM), jnp.float32),
        pltpu.VMEM((L, PITCH), jnp.float32),
        pltpu.VMEM((L, PITCH), jnp.float32),
        pltpu.VMEM((B_PER_W,), jnp.float32),
        pltpu.SemaphoreType.DMA((NBUF,)),
        pltpu.SemaphoreType.DMA((NBUF,)),
    ],
)
def _skipgram_sc(tgt_hbm, ctx_hbm, emb_hbm, ow_hbm, out_hbm,
                 tgt_v, ctx_v, e_buf, w_buf, stg_a, stg_b, out_v,
                 sem_e, sem_w):
    wid = lax.axis_index("s") * NC + lax.axis_index("c")
    base = wid * B_PER_W

    lane = lax.iota(jnp.int32, L)

    def copy_pair(k, p):
        return (
            pltpu.make_async_copy(
                emb_hbm.at[tgt_v.at[pl.ds(k * CHUNK, CHUNK)]],
                e_buf.at[p], sem_e.at[p]),
            pltpu.make_async_copy(
                ow_hbm.at[ctx_v.at[pl.ds(k * CHUNK, CHUNK)]],
                w_buf.at[p], sem_w.at[p]),
        )

    rest = B_PER_W - CHUNK
    it0 = pltpu.make_async_copy(
        tgt_hbm.at[pl.ds(base, CHUNK)], tgt_v.at[pl.ds(0, CHUNK)],
        sem_e.at[NBUF - 1])
    ic0 = pltpu.make_async_copy(
        ctx_hbm.at[pl.ds(base, CHUNK)], ctx_v.at[pl.ds(0, CHUNK)],
        sem_w.at[NBUF - 1])
    it0.start()
    ic0.start()
    itr = pltpu.make_async_copy(
        tgt_hbm.at[pl.ds(base + CHUNK, rest)], tgt_v.at[pl.ds(CHUNK, rest)],
        sem_e.at[NBUF - 2])
    icr = pltpu.make_async_copy(
        ctx_hbm.at[pl.ds(base + CHUNK, rest)], ctx_v.at[pl.ds(CHUNK, rest)],
        sem_w.at[NBUF - 2])
    itr.start()
    icr.start()
    it0.wait()
    ic0.wait()
    ce0, cw0 = copy_pair(0, 0)
    ce0.start()
    cw0.start()
    itr.wait()
    icr.wait()
    for k in range(1, min(NBUF, N_CHUNKS)):
        ce, cw = copy_pair(k, k)
        ce.start()
        cw.start()

    def chunk_step(k, carry):
        p = lax.rem(k, NBUF)
        ce, cw = copy_pair(k, p)
        ce.wait()
        cw.wait()

        e_b = e_buf.at[p]
        w_b = w_buf.at[p]

        def do_group(g, stg):
            rbase = g * L
            for sub in range(L // 4):
                paccs = []
                for rr in range(4):
                    row = rbase + sub * 4 + rr
                    prods = [e_b[row, pl.ds(c * L, L)] * w_b[row, pl.ds(c * L, L)]
                             for c in range(NVEC)]
                    while len(prods) > 1:
                        prods = [a + b for a, b in zip(prods[::2], prods[1::2])]
                    paccs.append(prods[0])
                for rr in range(4):
                    plsc.store_scatter(
                        stg,
                        [lane, jnp.full((L,), sub * 4 + rr, jnp.int32)],
                        paccs[rr])
            sums = [stg[j, pl.ds(0, L)] for j in range(L)]
            while len(sums) > 1:
                sums = [a + b for a, b in zip(sums[::2], sums[1::2])]
            out_v[pl.ds(k * CHUNK + rbase, L)] = sums[0]

        def group_pair(gp, carry2):
            do_group(gp * 2, stg_a)
            do_group(gp * 2 + 1, stg_b)
            return carry2

        lax.fori_loop(0, GROUPS // 2, group_pair, 0)

        @pl.when(k + NBUF < N_CHUNKS)
        def _():
            cne, cnw = copy_pair(k + NBUF, p)
            cne.start()
            cnw.start()

        return carry

    lax.fori_loop(0, N_CHUNKS, chunk_step, 0)

    pltpu.sync_copy(out_v, out_hbm.at[pl.ds(base, B_PER_W)])


def kernel(target, context, embeddings, output_weights):
    return _skipgram_sc(target, context, embeddings, output_weights)
